# baseline (speedup 1.0000x reference)
import functools

import jax
import jax.numpy as jnp
from jax import lax
from jax.experimental import pallas as pl
from jax.experimental.pallas import tpu as pltpu
from jax.experimental.pallas import tpu_sc as plsc

_LANES = 16
_NUM_WORKERS = 32
_NUM_CORES = 2


def _sc_body(b_per_w, c_dim, t_dim, x_hbm, idx_hbm, bank_hbm, out_hbm,
             idx_vm, nbuf, xbuf, obuf, sem_n, sem_x):
    wid = lax.axis_index("s") * _NUM_CORES + lax.axis_index("c")
    base = wid * b_per_w
    pltpu.sync_copy(idx_hbm.at[pl.ds(base, _LANES)], idx_vm)
    idxvec = idx_vm[...]

    n_full = t_dim // _LANES
    tail = t_dim - n_full * _LANES
    tail_off = t_dim - _LANES

    for j in range(b_per_w):
        row_idx = jnp.squeeze(lax.slice(idxvec, (j,), (j + 1,)))
        cp_n = pltpu.async_copy(bank_hbm.at[pl.ds(row_idx, 1)], nbuf, sem_n)
        cp_x = pltpu.async_copy(x_hbm.at[pl.ds(base + j, 1)], xbuf, sem_x)
        cp_n.wait()
        cp_x.wait()

        def chan(c, cc):
            def add(i, c2):
                sl = pl.ds(i * _LANES, _LANES)
                obuf[0, c, sl] = xbuf[0, c, sl] + nbuf[0, c, sl]
                return c2

            lax.fori_loop(0, n_full, add, 0, unroll=8)
            if tail:
                sl = pl.ds(tail_off, _LANES)
                obuf[0, c, sl] = xbuf[0, c, sl] + nbuf[0, c, sl]
            return cc

        lax.fori_loop(0, c_dim, chan, 0)
        pltpu.sync_copy(obuf, out_hbm.at[pl.ds(base + j, 1)])


@jax.jit
def kernel(x, indices, noise_bank):
    B, C, T = x.shape
    b_per_w = B // _NUM_WORKERS
    idx = jnp.concatenate(
        [indices.astype(jnp.int32), jnp.zeros((_LANES,), jnp.int32)])

    mesh = plsc.VectorSubcoreMesh(core_axis_name="c", subcore_axis_name="s")
    run = pl.kernel(
        functools.partial(_sc_body, b_per_w, C, T),
        out_type=jax.ShapeDtypeStruct((B, C, T), jnp.float32),
        scratch_types=[
            pltpu.VMEM((_LANES,), jnp.int32),
            pltpu.VMEM((1, C, T), jnp.float32),
            pltpu.VMEM((1, C, T), jnp.float32),
            pltpu.VMEM((1, C, T), jnp.float32),
            pltpu.SemaphoreType.DMA,
            pltpu.SemaphoreType.DMA,
        ],
        mesh=mesh,
    )
    return run(x, idx, noise_bank)


# speedup vs baseline: 2.8202x; 2.8202x over previous
import functools

import jax
import jax.numpy as jnp
from jax import lax
from jax.experimental import pallas as pl
from jax.experimental.pallas import tpu as pltpu
from jax.experimental.pallas import tpu_sc as plsc

_LANES = 16
_NUM_WORKERS = 32
_NUM_CORES = 2
_DT = 8


def _sc_body(c_dim, t_dim, n_bank, b_dim, x_hbm, idx_hbm, bank_hbm, out_hbm,
             idx_v, vb0, vb1, vb2, xb0, xb1, xb2, ob0, ob1, ob2,
             sem_v0, sem_v1, sem_v2, sem_x0, sem_x1, sem_x2,
             sem_o0, sem_o1, sem_o2):
    c = lax.axis_index("s") * _NUM_CORES + lax.axis_index("c")
    pltpu.sync_copy(idx_hbm, idx_v)
    n_chunks = b_dim // _LANES
    colv = [idx_v[pl.ds(k * _LANES, _LANES)] for k in range(n_chunks)]
    zero16 = jnp.zeros((_LANES,), jnp.int32)
    tvecs = [jnp.full((_LANES,), t, jnp.int32) for t in range(_DT)]
    n_slabs = t_dim // _DT

    vbs = (vb0, vb1, vb2)
    xbs = (xb0, xb1, xb2)
    obs = (ob0, ob1, ob2)
    sems_v = (sem_v0, sem_v1, sem_v2)
    sems_x = (sem_x0, sem_x1, sem_x2)
    sems_o = (sem_o0, sem_o1, sem_o2)

    def issue(s, r):
        t0 = pl.multiple_of(s * _DT, _DT)
        h = n_bank // 2
        pltpu.async_copy(
            bank_hbm.at[pl.ds(c, 1), pl.ds(t0, _DT), pl.ds(0, h)],
            vbs[r].at[:, :, pl.ds(0, h)], sems_v[r])
        pltpu.async_copy(
            bank_hbm.at[pl.ds(c, 1), pl.ds(t0, _DT), pl.ds(h, h)],
            vbs[r].at[:, :, pl.ds(h, h)], sems_v[r])
        pltpu.async_copy(
            x_hbm.at[pl.ds(c, 1), pl.ds(t0, _DT)], xbs[r], sems_x[r])

    def wait(r):
        pltpu.make_async_copy(
            bank_hbm.at[pl.ds(0, 1), pl.ds(0, _DT)], vbs[r], sems_v[r]).wait()
        pltpu.make_async_copy(
            x_hbm.at[pl.ds(0, 1), pl.ds(0, _DT)], xbs[r], sems_x[r]).wait()

    def wait_out(r):
        pltpu.make_async_copy(
            obs[r], out_hbm.at[pl.ds(0, 1), pl.ds(0, _DT)], sems_o[r]).wait()

    def compute_out(s, r):
        vb, xb, ob = vbs[r], xbs[r], obs[r]
        for t in range(_DT):
            for k in range(n_chunks):
                nv = plsc.load_gather(vb, [zero16, tvecs[t], colv[k]])
                sl = pl.ds(k * _LANES, _LANES)
                ob[0, t, sl] = xb[0, t, sl] + nv
        t0 = pl.multiple_of(s * _DT, _DT)
        pltpu.async_copy(ob, out_hbm.at[pl.ds(c, 1), pl.ds(t0, _DT)], sems_o[r])

    issue(0, 0)
    issue(1, 1)
    issue(2, 2)

    n_trips = (n_slabs - 2) // 3

    def trip(g, carry):
        s0 = 3 * g
        for r in range(3):
            wait(r)

            @pl.when(g > 0)
            def _(r=r):
                wait_out(r)

            compute_out(s0 + r, r)
            if r < 2:
                issue(s0 + r + 3, r)
            else:
                @pl.when(g < n_trips - 1)
                def _(r=r):
                    issue(s0 + r + 3, r)

        return carry

    lax.fori_loop(0, n_trips, trip, 0)
    wait(0)
    wait_out(0)
    compute_out(n_slabs - 2, 0)
    wait(1)
    wait_out(1)
    compute_out(n_slabs - 1, 1)
    wait_out(0)
    wait_out(1)
    wait_out(2)


_T_SC = 400
_TCHUNK = 200


def _tc_body(x_ref, bank_ref, oh_ref, out_ref):
    sel = jnp.dot(bank_ref[0], oh_ref[...],
                  preferred_element_type=jnp.float32)
    out_ref[0] = x_ref[0] + sel


@jax.jit
def kernel(x, indices, noise_bank):
    B, C, T = x.shape
    N = noise_bank.shape[0]
    assert C == _NUM_WORKERS
    assert _T_SC % _DT == 0 and (_T_SC // _DT) % 3 == 2
    assert _T_SC % _TCHUNK == 0 and (T - _T_SC) % _TCHUNK == 0

    xT = jnp.transpose(x, (1, 2, 0))
    bankT = jnp.transpose(noise_bank, (1, 2, 0))
    idx = indices.astype(jnp.int32)

    mesh = plsc.VectorSubcoreMesh(core_axis_name="c", subcore_axis_name="s")
    run = pl.kernel(
        functools.partial(_sc_body, C, _T_SC, N, B),
        out_type=jax.ShapeDtypeStruct((C, _T_SC, B), jnp.float32),
        scratch_types=(
            [pltpu.VMEM((B,), jnp.int32)]
            + [pltpu.VMEM((1, _DT, N), jnp.float32)] * 3
            + [pltpu.VMEM((1, _DT, B), jnp.float32)] * 6
            + [pltpu.SemaphoreType.DMA] * 9
        ),
        mesh=mesh,
        compiler_params=pltpu.CompilerParams(needs_layout_passes=False),
    )
    out_lo = run(xT, idx, bankT)

    onehot = (idx[None, :] == jnp.arange(N, dtype=jnp.int32)[:, None]
              ).astype(jnp.float32)
    toff = _T_SC // _TCHUNK
    out_hi = pl.pallas_call(
        _tc_body,
        grid=(C, (T - _T_SC) // _TCHUNK),
        in_specs=[
            pl.BlockSpec((1, _TCHUNK, B), lambda c, t: (c, t + toff, 0)),
            pl.BlockSpec((1, _TCHUNK, N), lambda c, t: (c, t + toff, 0)),
            pl.BlockSpec((N, B), lambda c, t: (0, 0)),
        ],
        out_specs=pl.BlockSpec((1, _TCHUNK, B), lambda c, t: (c, t, 0)),
        out_shape=jax.ShapeDtypeStruct((C, T - _T_SC, B), jnp.float32),
    )(xT, bankT, onehot)

    outT = jnp.concatenate([out_lo, out_hi], axis=1)
    return jnp.transpose(outT, (2, 0, 1))
